# baseline (speedup 1.0000x reference)
import jax
import jax.numpy as jnp
from jax import lax
from jax.experimental import pallas as pl
from jax.experimental.pallas import tpu as pltpu
from jax.experimental.pallas import tpu_sc as plsc

N = 10000
E = 320000
D = 128
H = 256
ED = 16
G = 100
NPG = 100
MLP_DIM = 512
C = 10

NC = 2
NS = 16
NW = NC * NS
K = 80
NP = 10240
RPT = NP // NS
L = 16


def _mesh():
    return plsc.VectorSubcoreMesh(core_axis_name="c", subcore_axis_name="s",
                                  num_cores=NC, num_subcores=NS)


_SC_PARAMS = pltpu.CompilerParams(use_tc_tiling_on_sc=False)


def _sc_phase_a(xt, ei, ea, ztd, zt16, onesk,
                outx, oute, outcnt,
                idxsA, idxsB, idxdA, idxdB, rowsA, rowsB, eaA, eaB, onesb,
                accx, acce, acccnt,
                ssA, ssB, sdA, sdB, sgA, sgB, seA, seB):
    c = lax.axis_index("c")
    s = lax.axis_index("s")
    wid = s * NC + c
    tb = s * RPT
    pltpu.sync_copy(ztd, accx.at[pl.ds(tb, RPT)])
    pltpu.sync_copy(zt16, acce.at[pl.ds(tb, RPT)])
    pltpu.sync_copy(zt16, acccnt.at[pl.ds(tb, RPT)])
    pltpu.sync_copy(onesk, onesb)

    nchunk = E // NW // K
    ebase = wid * (E // NW)

    def issue_idx(j, idxs, idxd, ss, sd):
        pltpu.async_copy(ei.at[0, pl.ds(ebase + j * K, K)], idxs, ss)
        pltpu.async_copy(ei.at[1, pl.ds(ebase + j * K, K)], idxd, sd)

    def wait_idx(j, idxs, idxd, ss, sd):
        pltpu.make_async_copy(ei.at[0, pl.ds(ebase + j * K, K)], idxs, ss).wait()
        pltpu.make_async_copy(ei.at[1, pl.ds(ebase + j * K, K)], idxd, sd).wait()

    def gather(j, idxs, rows, earows, sg, se):
        pltpu.async_copy(xt.at[idxs], rows, sg)
        pltpu.async_copy(ea.at[pl.ds(ebase + j * K, K)], earows, se)

    def scatter(j, idxs, idxd, rows, earows, sg, se):
        pltpu.make_async_copy(xt.at[idxs], rows, sg).wait()
        pltpu.sync_copy(rows, accx.at[idxd], add=True)
        pltpu.make_async_copy(ea.at[pl.ds(ebase + j * K, K)], earows, se).wait()
        pltpu.sync_copy(earows, acce.at[idxd], add=True)
        pltpu.sync_copy(onesb, acccnt.at[idxd], add=True)

    plsc.subcore_barrier()
    issue_idx(0, idxsA, idxdA, ssA, sdA)
    issue_idx(1, idxsB, idxdB, ssB, sdB)

    def body(i, carry):
        a = 2 * i
        b = a + 1
        wait_idx(a, idxsA, idxdA, ssA, sdA)
        gather(a, idxsA, rowsA, eaA, sgA, seA)

        @pl.when(b < nchunk)
        def _():
            wait_idx(b, idxsB, idxdB, ssB, sdB)
            gather(b, idxsB, rowsB, eaB, sgB, seB)
        scatter(a, idxsA, idxdA, rowsA, eaA, sgA, seA)

        @pl.when(a + 2 < nchunk)
        def _():
            issue_idx(a + 2, idxsA, idxdA, ssA, sdA)

        @pl.when(b < nchunk)
        def _():
            scatter(b, idxsB, idxdB, rowsB, eaB, sgB, seB)

            @pl.when(b + 2 < nchunk)
            def _():
                issue_idx(b + 2, idxsB, idxdB, ssB, sdB)
        return carry

    lax.fori_loop(0, (nchunk + 1) // 2, body, 0)
    plsc.subcore_barrier()
    pltpu.sync_copy(accx.at[pl.ds(tb, RPT)], outx.at[pl.ds(c * NP + tb, RPT)])
    pltpu.sync_copy(acce.at[pl.ds(tb, RPT)], oute.at[pl.ds(c * NP + tb, RPT)])
    pltpu.sync_copy(acccnt.at[pl.ds(tb, RPT)], outcnt.at[pl.ds(c * NP + tb, RPT)])


def _make_phase_a():
  return pl.kernel(
    _sc_phase_a,
    out_type=(jax.ShapeDtypeStruct((NC * NP, D), jnp.float32),
              jax.ShapeDtypeStruct((NC * NP, ED), jnp.float32),
              jax.ShapeDtypeStruct((NC * NP, ED), jnp.float32)),
    mesh=_mesh(),
    scratch_types=[
        pltpu.VMEM((K,), jnp.int32),
        pltpu.VMEM((K,), jnp.int32),
        pltpu.VMEM((K,), jnp.int32),
        pltpu.VMEM((K,), jnp.int32),
        pltpu.VMEM((K, D), jnp.float32),
        pltpu.VMEM((K, D), jnp.float32),
        pltpu.VMEM((K, ED), jnp.float32),
        pltpu.VMEM((K, ED), jnp.float32),
        pltpu.VMEM((K, ED), jnp.float32),
        pltpu.VMEM_SHARED((NP, D), jnp.float32),
        pltpu.VMEM_SHARED((NP, ED), jnp.float32),
        pltpu.VMEM_SHARED((NP, ED), jnp.float32),
    ] + [pltpu.SemaphoreType.DMA] * 8,
    compiler_params=_SC_PARAMS,
  )


def _sc_phase_b(h0a, h0b, ei, ztd,
                outh,
                idxsA, idxsB, idxdA, idxdB, rowsA, rowsB, acch,
                ssA, ssB, sdA, sdB, sgA, sgB):
    c = lax.axis_index("c")
    s = lax.axis_index("s")
    tb = s * RPT
    pltpu.sync_copy(ztd, acch.at[pl.ds(tb, RPT)])

    nchunk = E // NS // K
    ebase = s * (E // NS)

    def issue_idx(j, idxs, idxd, ss, sd):
        pltpu.async_copy(ei.at[0, pl.ds(ebase + j * K, K)], idxs, ss)
        pltpu.async_copy(ei.at[1, pl.ds(ebase + j * K, K)], idxd, sd)

    def wait_idx(j, idxs, idxd, ss, sd):
        pltpu.make_async_copy(ei.at[0, pl.ds(ebase + j * K, K)], idxs, ss).wait()
        pltpu.make_async_copy(ei.at[1, pl.ds(ebase + j * K, K)], idxd, sd).wait()

    def gather(idxs, rows, sg):
        @pl.when(c == 0)
        def _():
            pltpu.async_copy(h0a.at[idxs], rows, sg)

        @pl.when(c == 1)
        def _():
            pltpu.async_copy(h0b.at[idxs], rows, sg)

    def scatter(idxs, idxd, rows, sg):
        pltpu.make_async_copy(h0a.at[idxs], rows, sg).wait()
        pltpu.sync_copy(rows, acch.at[idxd], add=True)

    plsc.subcore_barrier()
    issue_idx(0, idxsA, idxdA, ssA, sdA)
    issue_idx(1, idxsB, idxdB, ssB, sdB)

    def body(i, carry):
        a = 2 * i
        b = a + 1
        wait_idx(a, idxsA, idxdA, ssA, sdA)
        gather(idxsA, rowsA, sgA)
        wait_idx(b, idxsB, idxdB, ssB, sdB)
        gather(idxsB, rowsB, sgB)
        scatter(idxsA, idxdA, rowsA, sgA)

        @pl.when(a + 2 < nchunk)
        def _():
            issue_idx(a + 2, idxsA, idxdA, ssA, sdA)
        scatter(idxsB, idxdB, rowsB, sgB)

        @pl.when(b + 2 < nchunk)
        def _():
            issue_idx(b + 2, idxsB, idxdB, ssB, sdB)
        return carry

    lax.fori_loop(0, nchunk // 2, body, 0)
    plsc.subcore_barrier()
    pltpu.sync_copy(acch.at[pl.ds(tb, RPT)], outh.at[pl.ds(c * NP + tb, RPT)])


def _make_phase_b():
  return pl.kernel(
    _sc_phase_b,
    out_type=jax.ShapeDtypeStruct((NC * NP, D), jnp.float32),
    mesh=_mesh(),
    scratch_types=[
        pltpu.VMEM((K,), jnp.int32),
        pltpu.VMEM((K,), jnp.int32),
        pltpu.VMEM((K,), jnp.int32),
        pltpu.VMEM((K,), jnp.int32),
        pltpu.VMEM((K, D), jnp.float32),
        pltpu.VMEM((K, D), jnp.float32),
        pltpu.VMEM_SHARED((NP, D), jnp.float32),
    ] + [pltpu.SemaphoreType.DMA] * 6,
    compiler_params=_SC_PARAMS,
  )


RB = 1000
NG = N // RB
GB = G // NG


def _k1_body(x_ref, aggx_ref, agge_ref, cnt_ref, w0d_ref, w0s_ref, w0e_ref,
             b0_ref, g_ref, be_ref, h0a_ref, h0b_ref, aux_ref, mp_ref, st_ref):
    p = pl.program_id(0)

    cnt = cnt_ref[0][:, 0:1] + cnt_ref[1][:, 0:1] + 1.0
    ea = agge_ref[0] + agge_ref[1]
    aux_ref[0] = jnp.broadcast_to(cnt, (RB, ED))
    aux_ref[1] = ea

    @pl.when(p == 0)
    def _():
        i = pl.program_id(1)
        x = x_ref[...]
        aggx = aggx_ref[0] + aggx_ref[1] + x
        w0e = w0e_ref[...]
        u = jnp.dot(x, w0d_ref[...], preferred_element_type=jnp.float32) + b0_ref[...]
        pre = (cnt * u
               + jnp.dot(aggx, w0s_ref[...], preferred_element_type=jnp.float32)
               + jnp.dot(ea, w0e, preferred_element_type=jnp.float32)
               + jnp.sum(w0e, axis=0, keepdims=True))
        mp = jnp.maximum(pre, 0.0)
        mp_ref[pl.ds(i * RB, RB)] = mp

        @pl.when(i == 0)
        def _():
            st_ref[...] = jnp.zeros_like(st_ref)
        st_ref[0:1] += jnp.sum(mp, axis=0, keepdims=True)
        st_ref[1:2] += jnp.sum(mp * mp, axis=0, keepdims=True)

    @pl.when(p == 1)
    def _():
        i = pl.program_id(1)
        mp = mp_ref[pl.ds(i * RB, RB)]
        mu = st_ref[0:1] * (1.0 / N)
        var = st_ref[1:2] * (1.0 / N) - mu * mu
        h = jnp.maximum((mp - mu) * lax.rsqrt(var + 1e-5) * g_ref[...]
                        + be_ref[...], 0.0)
        h0a_ref[...] = h[:, 0:D]
        h0b_ref[...] = h[:, D:2 * D]


_k1 = pl.pallas_call(
    _k1_body,
    grid=(2, NG),
    in_specs=[
        pl.BlockSpec((RB, D), lambda p, i: (jnp.where(p == 0, i, 0), 0)),
        pl.BlockSpec((2, RB, D), lambda p, i: (0, jnp.where(p == 0, i, 0), 0)),
        pl.BlockSpec((2, RB, ED), lambda p, i: (0, jnp.where(p == 0, i, 0), 0)),
        pl.BlockSpec((2, RB, ED), lambda p, i: (0, jnp.where(p == 0, i, 0), 0)),
        pl.BlockSpec((D, H), lambda p, i: (0, 0)),
        pl.BlockSpec((D, H), lambda p, i: (0, 0)),
        pl.BlockSpec((ED, H), lambda p, i: (0, 0)),
        pl.BlockSpec((1, H), lambda p, i: (0, 0)),
        pl.BlockSpec((1, H), lambda p, i: (0, 0)),
        pl.BlockSpec((1, H), lambda p, i: (0, 0)),
    ],
    out_specs=[
        pl.BlockSpec((RB, D), lambda p, i: (jnp.where(p == 1, i, 0), 0)),
        pl.BlockSpec((RB, D), lambda p, i: (jnp.where(p == 1, i, 0), 0)),
        pl.BlockSpec((2, RB, ED), lambda p, i: (0, jnp.where(p == 0, i, 0), 0)),
    ],
    out_shape=(jax.ShapeDtypeStruct((NP, D), jnp.float32),
               jax.ShapeDtypeStruct((NP, D), jnp.float32),
               jax.ShapeDtypeStruct((2, N, ED), jnp.float32)),
    scratch_shapes=[pltpu.VMEM((N, H), jnp.float32),
                    pltpu.VMEM((2, H), jnp.float32)],
)


def _k2_body(h0a_ref, h0b_ref, aggh_ref, aux_ref, w1d_ref, w1s_ref, w1e_ref,
             b1_ref, g_ref, be_ref, pooled_ref, ne0_ref, ne1_ref, mp_ref,
             st_ref):
    p = pl.program_id(0)

    @pl.when(p == 0)
    def _():
        i = pl.program_id(1)
        h0 = jnp.concatenate([h0a_ref[...], h0b_ref[...]], axis=1)
        aggh = jnp.concatenate([aggh_ref[0], aggh_ref[1]], axis=1) + h0
        cnt = aux_ref[0][:, 0:1]
        ea = aux_ref[1][...]
        w1e = w1e_ref[...]
        u = jnp.dot(h0, w1d_ref[...], preferred_element_type=jnp.float32) + b1_ref[...]
        pre = (cnt * u
               + jnp.dot(aggh, w1s_ref[...], preferred_element_type=jnp.float32)
               + jnp.dot(ea, w1e, preferred_element_type=jnp.float32)
               + jnp.sum(w1e, axis=0, keepdims=True))
        mp = jnp.maximum(pre, 0.0)
        mp_ref[pl.ds(i * RB, RB)] = mp

        @pl.when(i == 0)
        def _():
            st_ref[...] = jnp.zeros_like(st_ref)
        st_ref[0:1] += jnp.sum(mp, axis=0, keepdims=True)
        st_ref[1:2] += jnp.sum(mp * mp, axis=0, keepdims=True)

    @pl.when(p == 1)
    def _():
        i = pl.program_id(1)
        mp = mp_ref[pl.ds(i * RB, RB)]
        mu = st_ref[0:1] * (1.0 / N)
        var = st_ref[1:2] * (1.0 / N) - mu * mu
        h1 = jnp.maximum((mp - mu) * lax.rsqrt(var + 1e-5) * g_ref[...]
                         + be_ref[...], 0.0)
        h0 = jnp.concatenate([h0a_ref[...], h0b_ref[...]], axis=1)
        col = lax.broadcasted_iota(jnp.int32, (GB, RB), 1)
        row = lax.broadcasted_iota(jnp.int32, (GB, RB), 0)
        seg = jnp.where((col >= row * NPG) & (col < row * NPG + NPG), 1.0, 0.0)
        sel = jnp.where(col == row * NPG, 1.0, 0.0)
        pooled_ref[0] = jnp.dot(seg, h1, preferred_element_type=jnp.float32)
        ne0_ref[0] = jnp.dot(sel, h0, preferred_element_type=jnp.float32)
        ne1_ref[0] = jnp.dot(sel, h1, preferred_element_type=jnp.float32)


_k2 = pl.pallas_call(
    _k2_body,
    grid=(2, NG),
    in_specs=[
        pl.BlockSpec((RB, D), lambda p, i: (i, 0)),
        pl.BlockSpec((RB, D), lambda p, i: (i, 0)),
        pl.BlockSpec((2, RB, D), lambda p, i: (0, jnp.where(p == 0, i, 0), 0)),
        pl.BlockSpec((2, RB, ED), lambda p, i: (0, jnp.where(p == 0, i, 0), 0)),
        pl.BlockSpec((H, H), lambda p, i: (0, 0)),
        pl.BlockSpec((H, H), lambda p, i: (0, 0)),
        pl.BlockSpec((ED, H), lambda p, i: (0, 0)),
        pl.BlockSpec((1, H), lambda p, i: (0, 0)),
        pl.BlockSpec((1, H), lambda p, i: (0, 0)),
        pl.BlockSpec((1, H), lambda p, i: (0, 0)),
    ],
    out_specs=[
        pl.BlockSpec((1, GB, H), lambda p, i: (jnp.where(p == 1, i, 0), 0, 0)),
        pl.BlockSpec((1, GB, H), lambda p, i: (jnp.where(p == 1, i, 0), 0, 0)),
        pl.BlockSpec((1, GB, H), lambda p, i: (jnp.where(p == 1, i, 0), 0, 0)),
    ],
    out_shape=(jax.ShapeDtypeStruct((NG, GB, H), jnp.float32),
               jax.ShapeDtypeStruct((NG, GB, H), jnp.float32),
               jax.ShapeDtypeStruct((NG, GB, H), jnp.float32)),
    scratch_shapes=[pltpu.VMEM((N, H), jnp.float32),
                    pltpu.VMEM((2, H), jnp.float32)],
)


def _k3_body(pooled_ref, ne0_ref, ne1_ref, wf1_ref, bf1_ref, wf2_ref, bf2_ref,
             out_ref):
    z = jnp.concatenate([pooled_ref[...], ne0_ref[...], ne1_ref[...]], axis=1)
    z = jnp.maximum(jnp.dot(z, wf1_ref[...], preferred_element_type=jnp.float32)
                    + bf1_ref[...], 0.0)
    out_ref[...] = (jnp.dot(z, wf2_ref[...], preferred_element_type=jnp.float32)
                    + bf2_ref[...])


_k3 = pl.pallas_call(
    _k3_body,
    out_shape=jax.ShapeDtypeStruct((G, C), jnp.float32),
)


@jax.jit
def kernel(x, edge_index, edge_attr, batch, mask,
           W0, b0, g0, be0, W1, b1, g1, be1, Wf1, bf1, Wf2, bf2):
    ztd = jnp.zeros((RPT, D), jnp.float32)
    zt16 = jnp.zeros((RPT, ED), jnp.float32)
    onesk = jnp.ones((K, ED), jnp.float32)

    aggx, agge, cnt = _make_phase_a()(x, edge_index, edge_attr, ztd, zt16,
                                      onesk)

    h0a, h0b, aux = _k1(x, aggx.reshape(2, NP, D), agge.reshape(2, NP, ED),
                        cnt.reshape(2, NP, ED), W0[0:D], W0[D:2 * D], W0[2 * D:],
                        b0.reshape(1, H), g0.reshape(1, H), be0.reshape(1, H))

    aggh = _make_phase_b()(h0a, h0b, edge_index, ztd)

    pooled, ne0, ne1 = _k2(h0a, h0b, aggh.reshape(2, NP, D), aux,
                           W1[0:H], W1[H:2 * H], W1[2 * H:],
                           b1.reshape(1, H), g1.reshape(1, H),
                           be1.reshape(1, H))
    return _k3(pooled.reshape(G, H), ne0.reshape(G, H), ne1.reshape(G, H),
               Wf1, bf1.reshape(1, MLP_DIM), Wf2, bf2.reshape(1, C))


# speedup vs baseline: 1.0251x; 1.0251x over previous
import jax
import jax.numpy as jnp
from jax import lax
from jax.experimental import pallas as pl
from jax.experimental.pallas import tpu as pltpu
from jax.experimental.pallas import tpu_sc as plsc

N = 10000
E = 320000
D = 128
H = 256
ED = 16
G = 100
NPG = 100
MLP_DIM = 512
C = 10

NC = 2
NS = 16
NW = NC * NS
K = 80
NP = 10240
RPT = NP // NS
L = 16


def _mesh():
    return plsc.VectorSubcoreMesh(core_axis_name="c", subcore_axis_name="s",
                                  num_cores=NC, num_subcores=NS)


_SC_PARAMS = pltpu.CompilerParams(use_tc_tiling_on_sc=False)


def _sc_phase_a(xt, ei, ea, ztd, zt16, onesk,
                outx, oute, outcnt,
                idxsA, idxsB, idxdA, idxdB, rowsA, rowsB, eaA, eaB, onesb,
                accx, acce, acccnt,
                ssA, ssB, sdA, sdB, sgA, sgB, seA, seB):
    c = lax.axis_index("c")
    s = lax.axis_index("s")
    wid = s * NC + c
    tb = s * RPT
    pltpu.sync_copy(ztd, accx.at[pl.ds(tb, RPT)])
    pltpu.sync_copy(zt16, acce.at[pl.ds(tb, RPT)])
    pltpu.sync_copy(zt16, acccnt.at[pl.ds(tb, RPT)])
    pltpu.sync_copy(onesk, onesb)

    nchunk = E // NW // K
    ebase = wid * (E // NW)

    def issue_idx(j, idxs, idxd, ss, sd):
        pltpu.async_copy(ei.at[0, pl.ds(ebase + j * K, K)], idxs, ss)
        pltpu.async_copy(ei.at[1, pl.ds(ebase + j * K, K)], idxd, sd)

    def wait_idx(j, idxs, idxd, ss, sd):
        pltpu.make_async_copy(ei.at[0, pl.ds(ebase + j * K, K)], idxs, ss).wait()
        pltpu.make_async_copy(ei.at[1, pl.ds(ebase + j * K, K)], idxd, sd).wait()

    def gather(j, idxs, rows, earows, sg, se):
        pltpu.async_copy(xt.at[idxs], rows, sg)
        pltpu.async_copy(ea.at[pl.ds(ebase + j * K, K)], earows, se)

    def scatter(j, idxs, idxd, rows, earows, sg, se):
        pltpu.make_async_copy(xt.at[idxs], rows, sg).wait()
        pltpu.sync_copy(rows, accx.at[idxd], add=True)
        pltpu.make_async_copy(ea.at[pl.ds(ebase + j * K, K)], earows, se).wait()
        pltpu.sync_copy(earows, acce.at[idxd], add=True)
        pltpu.sync_copy(onesb, acccnt.at[idxd], add=True)

    plsc.subcore_barrier()
    issue_idx(0, idxsA, idxdA, ssA, sdA)
    issue_idx(1, idxsB, idxdB, ssB, sdB)

    def body(i, carry):
        a = 2 * i
        b = a + 1
        wait_idx(a, idxsA, idxdA, ssA, sdA)
        gather(a, idxsA, rowsA, eaA, sgA, seA)

        @pl.when(b < nchunk)
        def _():
            wait_idx(b, idxsB, idxdB, ssB, sdB)
            gather(b, idxsB, rowsB, eaB, sgB, seB)
        scatter(a, idxsA, idxdA, rowsA, eaA, sgA, seA)

        @pl.when(a + 2 < nchunk)
        def _():
            issue_idx(a + 2, idxsA, idxdA, ssA, sdA)

        @pl.when(b < nchunk)
        def _():
            scatter(b, idxsB, idxdB, rowsB, eaB, sgB, seB)

            @pl.when(b + 2 < nchunk)
            def _():
                issue_idx(b + 2, idxsB, idxdB, ssB, sdB)
        return carry

    lax.fori_loop(0, (nchunk + 1) // 2, body, 0)
    plsc.subcore_barrier()
    pltpu.sync_copy(accx.at[pl.ds(tb, RPT)], outx.at[pl.ds(c * NP + tb, RPT)])
    pltpu.sync_copy(acce.at[pl.ds(tb, RPT)], oute.at[pl.ds(c * NP + tb, RPT)])
    pltpu.sync_copy(acccnt.at[pl.ds(tb, RPT)], outcnt.at[pl.ds(c * NP + tb, RPT)])


def _make_phase_a():
  return pl.kernel(
    _sc_phase_a,
    out_type=(jax.ShapeDtypeStruct((NC * NP, D), jnp.float32),
              jax.ShapeDtypeStruct((NC * NP, ED), jnp.float32),
              jax.ShapeDtypeStruct((NC * NP, ED), jnp.float32)),
    mesh=_mesh(),
    scratch_types=[
        pltpu.VMEM((K,), jnp.int32),
        pltpu.VMEM((K,), jnp.int32),
        pltpu.VMEM((K,), jnp.int32),
        pltpu.VMEM((K,), jnp.int32),
        pltpu.VMEM((K, D), jnp.float32),
        pltpu.VMEM((K, D), jnp.float32),
        pltpu.VMEM((K, ED), jnp.float32),
        pltpu.VMEM((K, ED), jnp.float32),
        pltpu.VMEM((K, ED), jnp.float32),
        pltpu.VMEM_SHARED((NP, D), jnp.float32),
        pltpu.VMEM_SHARED((NP, ED), jnp.float32),
        pltpu.VMEM_SHARED((NP, ED), jnp.float32),
    ] + [pltpu.SemaphoreType.DMA] * 8,
    compiler_params=_SC_PARAMS,
  )


def _sc_phase_b(h0a, h0b, ei, ztd,
                outh,
                idxsA, idxsB, idxdA, idxdB, rowsA, rowsB, acch,
                ssA, ssB, sdA, sdB, sgA, sgB):
    c = lax.axis_index("c")
    s = lax.axis_index("s")
    tb = s * RPT
    pltpu.sync_copy(ztd, acch.at[pl.ds(tb, RPT)])

    nchunk = E // NS // K
    ebase = s * (E // NS)

    def issue_idx(j, idxs, idxd, ss, sd):
        pltpu.async_copy(ei.at[0, pl.ds(ebase + j * K, K)], idxs, ss)
        pltpu.async_copy(ei.at[1, pl.ds(ebase + j * K, K)], idxd, sd)

    def wait_idx(j, idxs, idxd, ss, sd):
        pltpu.make_async_copy(ei.at[0, pl.ds(ebase + j * K, K)], idxs, ss).wait()
        pltpu.make_async_copy(ei.at[1, pl.ds(ebase + j * K, K)], idxd, sd).wait()

    def gather(idxs, rows, sg):
        @pl.when(c == 0)
        def _():
            pltpu.async_copy(h0a.at[idxs], rows, sg)

        @pl.when(c == 1)
        def _():
            pltpu.async_copy(h0b.at[idxs], rows, sg)

    def scatter(idxs, idxd, rows, sg):
        pltpu.make_async_copy(h0a.at[idxs], rows, sg).wait()
        pltpu.sync_copy(rows, acch.at[idxd], add=True)

    plsc.subcore_barrier()
    issue_idx(0, idxsA, idxdA, ssA, sdA)
    issue_idx(1, idxsB, idxdB, ssB, sdB)

    def body(i, carry):
        a = 2 * i
        b = a + 1
        wait_idx(a, idxsA, idxdA, ssA, sdA)
        gather(idxsA, rowsA, sgA)
        wait_idx(b, idxsB, idxdB, ssB, sdB)
        gather(idxsB, rowsB, sgB)
        scatter(idxsA, idxdA, rowsA, sgA)

        @pl.when(a + 2 < nchunk)
        def _():
            issue_idx(a + 2, idxsA, idxdA, ssA, sdA)
        scatter(idxsB, idxdB, rowsB, sgB)

        @pl.when(b + 2 < nchunk)
        def _():
            issue_idx(b + 2, idxsB, idxdB, ssB, sdB)
        return carry

    lax.fori_loop(0, nchunk // 2, body, 0)
    plsc.subcore_barrier()
    pltpu.sync_copy(acch.at[pl.ds(tb, RPT)], outh.at[pl.ds(c * NP + tb, RPT)])


def _make_phase_b():
  return pl.kernel(
    _sc_phase_b,
    out_type=jax.ShapeDtypeStruct((NC * NP, D), jnp.float32),
    mesh=_mesh(),
    scratch_types=[
        pltpu.VMEM((K,), jnp.int32),
        pltpu.VMEM((K,), jnp.int32),
        pltpu.VMEM((K,), jnp.int32),
        pltpu.VMEM((K,), jnp.int32),
        pltpu.VMEM((K, D), jnp.float32),
        pltpu.VMEM((K, D), jnp.float32),
        pltpu.VMEM_SHARED((NP, D), jnp.float32),
    ] + [pltpu.SemaphoreType.DMA] * 6,
    compiler_params=_SC_PARAMS,
  )


RB = 2000
NG = N // RB
GB = G // NG


def _k1_body(x_ref, aggx_ref, agge_ref, cnt_ref, w0d_ref, w0s_ref, w0e_ref,
             b0_ref, g_ref, be_ref, h0a_ref, h0b_ref, aux_ref, mp_ref, st_ref):
    p = pl.program_id(0)

    cnt = cnt_ref[0][:, 0:1] + cnt_ref[1][:, 0:1] + 1.0
    ea = agge_ref[0] + agge_ref[1]
    aux_ref[0] = jnp.broadcast_to(cnt, (RB, ED))
    aux_ref[1] = ea

    @pl.when(p == 0)
    def _():
        i = pl.program_id(1)
        x = x_ref[...]
        aggx = aggx_ref[0] + aggx_ref[1] + x
        w0e = w0e_ref[...]
        u = jnp.dot(x, w0d_ref[...], preferred_element_type=jnp.float32) + b0_ref[...]
        pre = (cnt * u
               + jnp.dot(aggx, w0s_ref[...], preferred_element_type=jnp.float32)
               + jnp.dot(ea, w0e, preferred_element_type=jnp.float32)
               + jnp.sum(w0e, axis=0, keepdims=True))
        mp = jnp.maximum(pre, 0.0)
        mp_ref[pl.ds(i * RB, RB)] = mp

        @pl.when(i == 0)
        def _():
            st_ref[...] = jnp.zeros_like(st_ref)
        st_ref[0:1] += jnp.sum(mp, axis=0, keepdims=True)
        st_ref[1:2] += jnp.sum(mp * mp, axis=0, keepdims=True)

    @pl.when(p == 1)
    def _():
        i = pl.program_id(1)
        mp = mp_ref[pl.ds(i * RB, RB)]
        mu = st_ref[0:1] * (1.0 / N)
        var = st_ref[1:2] * (1.0 / N) - mu * mu
        h = jnp.maximum((mp - mu) * lax.rsqrt(var + 1e-5) * g_ref[...]
                        + be_ref[...], 0.0)
        h0a_ref[...] = h[:, 0:D]
        h0b_ref[...] = h[:, D:2 * D]


_k1 = pl.pallas_call(
    _k1_body,
    grid=(2, NG),
    in_specs=[
        pl.BlockSpec((RB, D), lambda p, i: (jnp.where(p == 0, i, 0), 0)),
        pl.BlockSpec((2, RB, D), lambda p, i: (0, jnp.where(p == 0, i, 0), 0)),
        pl.BlockSpec((2, RB, ED), lambda p, i: (0, jnp.where(p == 0, i, 0), 0)),
        pl.BlockSpec((2, RB, ED), lambda p, i: (0, jnp.where(p == 0, i, 0), 0)),
        pl.BlockSpec((D, H), lambda p, i: (0, 0)),
        pl.BlockSpec((D, H), lambda p, i: (0, 0)),
        pl.BlockSpec((ED, H), lambda p, i: (0, 0)),
        pl.BlockSpec((1, H), lambda p, i: (0, 0)),
        pl.BlockSpec((1, H), lambda p, i: (0, 0)),
        pl.BlockSpec((1, H), lambda p, i: (0, 0)),
    ],
    out_specs=[
        pl.BlockSpec((RB, D), lambda p, i: (jnp.where(p == 1, i, 0), 0)),
        pl.BlockSpec((RB, D), lambda p, i: (jnp.where(p == 1, i, 0), 0)),
        pl.BlockSpec((2, RB, ED), lambda p, i: (0, jnp.where(p == 0, i, 0), 0)),
    ],
    out_shape=(jax.ShapeDtypeStruct((NP, D), jnp.float32),
               jax.ShapeDtypeStruct((NP, D), jnp.float32),
               jax.ShapeDtypeStruct((2, N, ED), jnp.float32)),
    scratch_shapes=[pltpu.VMEM((N, H), jnp.float32),
                    pltpu.VMEM((2, H), jnp.float32)],
)


def _k2_body(h0a_ref, h0b_ref, aggh_ref, aux_ref, w1d_ref, w1s_ref, w1e_ref,
             b1_ref, g_ref, be_ref, wf1_ref, bf1_ref, wf2_ref, bf2_ref,
             out_ref, mp_ref, st_ref, z_ref):
    p = pl.program_id(0)

    @pl.when(p == 0)
    def _():
        i = pl.program_id(1)
        h0 = jnp.concatenate([h0a_ref[...], h0b_ref[...]], axis=1)
        aggh = jnp.concatenate([aggh_ref[0], aggh_ref[1]], axis=1) + h0
        cnt = aux_ref[0][:, 0:1]
        ea = aux_ref[1][...]
        w1e = w1e_ref[...]
        u = jnp.dot(h0, w1d_ref[...], preferred_element_type=jnp.float32) + b1_ref[...]
        pre = (cnt * u
               + jnp.dot(aggh, w1s_ref[...], preferred_element_type=jnp.float32)
               + jnp.dot(ea, w1e, preferred_element_type=jnp.float32)
               + jnp.sum(w1e, axis=0, keepdims=True))
        mp = jnp.maximum(pre, 0.0)
        mp_ref[pl.ds(i * RB, RB)] = mp

        @pl.when(i == 0)
        def _():
            st_ref[...] = jnp.zeros_like(st_ref)
        st_ref[0:1] += jnp.sum(mp, axis=0, keepdims=True)
        st_ref[1:2] += jnp.sum(mp * mp, axis=0, keepdims=True)

    @pl.when(p == 1)
    def _():
        i = pl.program_id(1)
        mp = mp_ref[pl.ds(i * RB, RB)]
        mu = st_ref[0:1] * (1.0 / N)
        var = st_ref[1:2] * (1.0 / N) - mu * mu
        h1 = jnp.maximum((mp - mu) * lax.rsqrt(var + 1e-5) * g_ref[...]
                         + be_ref[...], 0.0)
        h0 = jnp.concatenate([h0a_ref[...], h0b_ref[...]], axis=1)
        col = lax.broadcasted_iota(jnp.int32, (GB, RB), 1)
        row = lax.broadcasted_iota(jnp.int32, (GB, RB), 0)
        seg = jnp.where((col >= row * NPG) & (col < row * NPG + NPG), 1.0, 0.0)
        sel = jnp.where(col == row * NPG, 1.0, 0.0)
        z_ref[i] = jnp.concatenate(
            [jnp.dot(seg, h1, preferred_element_type=jnp.float32),
             jnp.dot(sel, h0, preferred_element_type=jnp.float32),
             jnp.dot(sel, h1, preferred_element_type=jnp.float32)], axis=1)

        @pl.when(i == NG - 1)
        def _():
            zz = jnp.maximum(
                jnp.dot(z_ref[...].reshape(G, 3 * H), wf1_ref[...],
                        preferred_element_type=jnp.float32) + bf1_ref[...], 0.0)
            out_ref[...] = (jnp.dot(zz, wf2_ref[...],
                                    preferred_element_type=jnp.float32)
                            + bf2_ref[...])


_k2 = pl.pallas_call(
    _k2_body,
    grid=(2, NG),
    in_specs=[
        pl.BlockSpec((RB, D), lambda p, i: (i, 0)),
        pl.BlockSpec((RB, D), lambda p, i: (i, 0)),
        pl.BlockSpec((2, RB, D), lambda p, i: (0, jnp.where(p == 0, i, 0), 0)),
        pl.BlockSpec((2, RB, ED), lambda p, i: (0, jnp.where(p == 0, i, 0), 0)),
        pl.BlockSpec((H, H), lambda p, i: (0, 0)),
        pl.BlockSpec((H, H), lambda p, i: (0, 0)),
        pl.BlockSpec((ED, H), lambda p, i: (0, 0)),
        pl.BlockSpec((1, H), lambda p, i: (0, 0)),
        pl.BlockSpec((1, H), lambda p, i: (0, 0)),
        pl.BlockSpec((1, H), lambda p, i: (0, 0)),
        pl.BlockSpec((H + 2 * H, MLP_DIM), lambda p, i: (0, 0)),
        pl.BlockSpec((1, MLP_DIM), lambda p, i: (0, 0)),
        pl.BlockSpec((MLP_DIM, C), lambda p, i: (0, 0)),
        pl.BlockSpec((1, C), lambda p, i: (0, 0)),
    ],
    out_specs=pl.BlockSpec((G, C), lambda p, i: (0, 0)),
    out_shape=jax.ShapeDtypeStruct((G, C), jnp.float32),
    scratch_shapes=[pltpu.VMEM((N, H), jnp.float32),
                    pltpu.VMEM((2, H), jnp.float32),
                    pltpu.VMEM((NG, GB, 3 * H), jnp.float32)],
)


def _k3_body(pooled_ref, ne0_ref, ne1_ref, wf1_ref, bf1_ref, wf2_ref, bf2_ref,
             out_ref):
    z = jnp.concatenate([pooled_ref[...], ne0_ref[...], ne1_ref[...]], axis=1)
    z = jnp.maximum(jnp.dot(z, wf1_ref[...], preferred_element_type=jnp.float32)
                    + bf1_ref[...], 0.0)
    out_ref[...] = (jnp.dot(z, wf2_ref[...], preferred_element_type=jnp.float32)
                    + bf2_ref[...])


_k3 = pl.pallas_call(
    _k3_body,
    out_shape=jax.ShapeDtypeStruct((G, C), jnp.float32),
)


@jax.jit
def kernel(x, edge_index, edge_attr, batch, mask,
           W0, b0, g0, be0, W1, b1, g1, be1, Wf1, bf1, Wf2, bf2):
    ztd = jnp.zeros((RPT, D), jnp.float32)
    zt16 = jnp.zeros((RPT, ED), jnp.float32)
    onesk = jnp.ones((K, ED), jnp.float32)

    aggx, agge, cnt = _make_phase_a()(x, edge_index, edge_attr, ztd, zt16,
                                      onesk)

    h0a, h0b, aux = _k1(x, aggx.reshape(2, NP, D), agge.reshape(2, NP, ED),
                        cnt.reshape(2, NP, ED), W0[0:D], W0[D:2 * D], W0[2 * D:],
                        b0.reshape(1, H), g0.reshape(1, H), be0.reshape(1, H))

    aggh = _make_phase_b()(h0a, h0b, edge_index, ztd)

    return _k2(h0a, h0b, aggh.reshape(2, NP, D), aux,
               W1[0:H], W1[H:2 * H], W1[2 * H:],
               b1.reshape(1, H), g1.reshape(1, H), be1.reshape(1, H),
               Wf1, bf1.reshape(1, MLP_DIM), Wf2, bf2.reshape(1, C))


# speedup vs baseline: 1.0267x; 1.0015x over previous
import jax
import jax.numpy as jnp
from jax import lax
from jax.experimental import pallas as pl
from jax.experimental.pallas import tpu as pltpu
from jax.experimental.pallas import tpu_sc as plsc

N = 10000
E = 320000
D = 128
H = 256
ED = 16
G = 100
NPG = 100
MLP_DIM = 512
C = 10

NC = 2
NS = 16
NW = NC * NS
K = 80
NP = 10240
RPT = NP // NS
L = 16


def _mesh():
    return plsc.VectorSubcoreMesh(core_axis_name="c", subcore_axis_name="s",
                                  num_cores=NC, num_subcores=NS)


_SC_PARAMS = pltpu.CompilerParams(use_tc_tiling_on_sc=False)


def _sc_phase_a(xt, ei, ea, ztd, zt16, onesk,
                outx, oute, outcnt,
                idxsA, idxsB, idxdA, idxdB, rowsA, rowsB, eaA, eaB, onesb,
                accx, acce, acccnt,
                ssA, ssB, sdA, sdB, sgA, sgB, seA, seB):
    c = lax.axis_index("c")
    s = lax.axis_index("s")
    wid = s * NC + c
    tb = s * RPT
    pltpu.sync_copy(ztd, accx.at[pl.ds(tb, RPT)])
    pltpu.sync_copy(zt16, acce.at[pl.ds(tb, RPT)])
    pltpu.sync_copy(zt16, acccnt.at[pl.ds(tb, RPT)])
    pltpu.sync_copy(onesk, onesb)

    nchunk = E // NW // K
    ebase = wid * (E // NW)

    def issue_idx(j, idxs, idxd, ss, sd):
        pltpu.async_copy(ei.at[0, pl.ds(ebase + j * K, K)], idxs, ss)
        pltpu.async_copy(ei.at[1, pl.ds(ebase + j * K, K)], idxd, sd)

    def wait_idx(j, idxs, idxd, ss, sd):
        pltpu.make_async_copy(ei.at[0, pl.ds(ebase + j * K, K)], idxs, ss).wait()
        pltpu.make_async_copy(ei.at[1, pl.ds(ebase + j * K, K)], idxd, sd).wait()

    def gather(j, idxs, rows, earows, sg, se):
        pltpu.async_copy(xt.at[idxs], rows, sg)
        pltpu.async_copy(ea.at[pl.ds(ebase + j * K, K)], earows, se)

    def scatter(j, idxs, idxd, rows, earows, sg, se):
        pltpu.make_async_copy(xt.at[idxs], rows, sg).wait()
        pltpu.sync_copy(rows, accx.at[idxd], add=True)
        pltpu.make_async_copy(ea.at[pl.ds(ebase + j * K, K)], earows, se).wait()
        pltpu.sync_copy(earows, acce.at[idxd], add=True)
        pltpu.sync_copy(onesb, acccnt.at[idxd], add=True)

    plsc.subcore_barrier()
    issue_idx(0, idxsA, idxdA, ssA, sdA)
    issue_idx(1, idxsB, idxdB, ssB, sdB)

    def body(i, carry):
        a = 2 * i
        b = a + 1
        wait_idx(a, idxsA, idxdA, ssA, sdA)
        gather(a, idxsA, rowsA, eaA, sgA, seA)

        @pl.when(b < nchunk)
        def _():
            wait_idx(b, idxsB, idxdB, ssB, sdB)
            gather(b, idxsB, rowsB, eaB, sgB, seB)
        scatter(a, idxsA, idxdA, rowsA, eaA, sgA, seA)

        @pl.when(a + 2 < nchunk)
        def _():
            issue_idx(a + 2, idxsA, idxdA, ssA, sdA)

        @pl.when(b < nchunk)
        def _():
            scatter(b, idxsB, idxdB, rowsB, eaB, sgB, seB)

            @pl.when(b + 2 < nchunk)
            def _():
                issue_idx(b + 2, idxsB, idxdB, ssB, sdB)
        return carry

    lax.fori_loop(0, (nchunk + 1) // 2, body, 0)
    plsc.subcore_barrier()
    pltpu.sync_copy(accx.at[pl.ds(tb, RPT)], outx.at[pl.ds(c * NP + tb, RPT)])
    pltpu.sync_copy(acce.at[pl.ds(tb, RPT)], oute.at[pl.ds(c * NP + tb, RPT)])
    pltpu.sync_copy(acccnt.at[pl.ds(tb, RPT)], outcnt.at[pl.ds(c * NP + tb, RPT)])


def _make_phase_a():
  return pl.kernel(
    _sc_phase_a,
    out_type=(jax.ShapeDtypeStruct((NC * NP, D), jnp.float32),
              jax.ShapeDtypeStruct((NC * NP, ED), jnp.float32),
              jax.ShapeDtypeStruct((NC * NP, ED), jnp.float32)),
    mesh=_mesh(),
    scratch_types=[
        pltpu.VMEM((K,), jnp.int32),
        pltpu.VMEM((K,), jnp.int32),
        pltpu.VMEM((K,), jnp.int32),
        pltpu.VMEM((K,), jnp.int32),
        pltpu.VMEM((K, D), jnp.float32),
        pltpu.VMEM((K, D), jnp.float32),
        pltpu.VMEM((K, ED), jnp.float32),
        pltpu.VMEM((K, ED), jnp.float32),
        pltpu.VMEM((K, ED), jnp.float32),
        pltpu.VMEM_SHARED((NP, D), jnp.float32),
        pltpu.VMEM_SHARED((NP, ED), jnp.float32),
        pltpu.VMEM_SHARED((NP, ED), jnp.float32),
    ] + [pltpu.SemaphoreType.DMA] * 8,
    compiler_params=_SC_PARAMS,
  )


def _sc_phase_b(h0a, h0b, ei, ztd,
                outh,
                idxsA, idxsB, idxdA, idxdB, rowsA, rowsB, acch,
                ssA, ssB, sdA, sdB, sgA, sgB):
    c = lax.axis_index("c")
    s = lax.axis_index("s")
    tb = s * RPT
    pltpu.sync_copy(ztd, acch.at[pl.ds(tb, RPT)])

    nchunk = E // NS // K
    ebase = s * (E // NS)

    def issue_idx(j, idxs, idxd, ss, sd):
        pltpu.async_copy(ei.at[0, pl.ds(ebase + j * K, K)], idxs, ss)
        pltpu.async_copy(ei.at[1, pl.ds(ebase + j * K, K)], idxd, sd)

    def wait_idx(j, idxs, idxd, ss, sd):
        pltpu.make_async_copy(ei.at[0, pl.ds(ebase + j * K, K)], idxs, ss).wait()
        pltpu.make_async_copy(ei.at[1, pl.ds(ebase + j * K, K)], idxd, sd).wait()

    def gather(idxs, rows, sg):
        @pl.when(c == 0)
        def _():
            pltpu.async_copy(h0a.at[idxs], rows, sg)

        @pl.when(c == 1)
        def _():
            pltpu.async_copy(h0b.at[idxs], rows, sg)

    def scatter(idxs, idxd, rows, sg):
        pltpu.make_async_copy(h0a.at[idxs], rows, sg).wait()
        pltpu.sync_copy(rows, acch.at[idxd], add=True)

    plsc.subcore_barrier()
    issue_idx(0, idxsA, idxdA, ssA, sdA)
    issue_idx(1, idxsB, idxdB, ssB, sdB)

    def body(i, carry):
        a = 2 * i
        b = a + 1
        wait_idx(a, idxsA, idxdA, ssA, sdA)
        gather(idxsA, rowsA, sgA)
        wait_idx(b, idxsB, idxdB, ssB, sdB)
        gather(idxsB, rowsB, sgB)
        scatter(idxsA, idxdA, rowsA, sgA)

        @pl.when(a + 2 < nchunk)
        def _():
            issue_idx(a + 2, idxsA, idxdA, ssA, sdA)
        scatter(idxsB, idxdB, rowsB, sgB)

        @pl.when(b + 2 < nchunk)
        def _():
            issue_idx(b + 2, idxsB, idxdB, ssB, sdB)
        return carry

    lax.fori_loop(0, nchunk // 2, body, 0)
    plsc.subcore_barrier()
    pltpu.sync_copy(acch.at[pl.ds(tb, RPT)], outh.at[pl.ds(c * NP + tb, RPT)])


def _make_phase_b():
  return pl.kernel(
    _sc_phase_b,
    out_type=jax.ShapeDtypeStruct((NC * NP, D), jnp.float32),
    mesh=_mesh(),
    scratch_types=[
        pltpu.VMEM((K,), jnp.int32),
        pltpu.VMEM((K,), jnp.int32),
        pltpu.VMEM((K,), jnp.int32),
        pltpu.VMEM((K,), jnp.int32),
        pltpu.VMEM((K, D), jnp.float32),
        pltpu.VMEM((K, D), jnp.float32),
        pltpu.VMEM_SHARED((NP, D), jnp.float32),
    ] + [pltpu.SemaphoreType.DMA] * 6,
    compiler_params=_SC_PARAMS,
  )


RB = 2000
NG = N // RB
GB = G // NG


def _k1_body(x_ref, aggx_ref, agge_ref, cnt_ref, w0d_ref, w0s_ref, w0e_ref,
             b0_ref, g_ref, be_ref, h0a_ref, h0b_ref, aux_ref, mp_ref, st_ref):
    p = pl.program_id(0)

    cnt = cnt_ref[0][:, 0:1] + cnt_ref[1][:, 0:1] + 1.0
    ea = agge_ref[0] + agge_ref[1]
    aux_ref[0] = jnp.broadcast_to(cnt, (RB, ED))
    aux_ref[1] = ea

    @pl.when(p == 0)
    def _():
        i = pl.program_id(1)
        x = x_ref[...]
        aggx = aggx_ref[0] + aggx_ref[1] + x
        w0e = w0e_ref[...]
        u = jnp.dot(x, w0d_ref[...], preferred_element_type=jnp.float32) + b0_ref[...]
        pre = (cnt * u
               + jnp.dot(aggx, w0s_ref[...], preferred_element_type=jnp.float32)
               + jnp.dot(ea, w0e, preferred_element_type=jnp.float32)
               + jnp.sum(w0e, axis=0, keepdims=True))
        mp = jnp.maximum(pre, 0.0)
        mp_ref[pl.ds(i * RB, RB)] = mp

        @pl.when(i == 0)
        def _():
            st_ref[...] = jnp.zeros_like(st_ref)
        st_ref[0:1] += jnp.sum(mp, axis=0, keepdims=True)
        st_ref[1:2] += jnp.sum(mp * mp, axis=0, keepdims=True)

    @pl.when(p == 1)
    def _():
        i = pl.program_id(1)
        mp = mp_ref[pl.ds(i * RB, RB)]
        mu = st_ref[0:1] * (1.0 / N)
        var = st_ref[1:2] * (1.0 / N) - mu * mu
        h = jnp.maximum((mp - mu) * lax.rsqrt(var + 1e-5) * g_ref[...]
                        + be_ref[...], 0.0)
        h0a_ref[...] = h[:, 0:D]
        h0b_ref[...] = h[:, D:2 * D]


_k1 = pl.pallas_call(
    _k1_body,
    grid=(2, NG),
    in_specs=[
        pl.BlockSpec((RB, D), lambda p, i: (jnp.where(p == 0, i, 0), 0)),
        pl.BlockSpec((2, RB, D), lambda p, i: (0, jnp.where(p == 0, i, 0), 0)),
        pl.BlockSpec((2, RB, ED), lambda p, i: (0, jnp.where(p == 0, i, 0), 0)),
        pl.BlockSpec((2, RB, ED), lambda p, i: (0, jnp.where(p == 0, i, 0), 0)),
        pl.BlockSpec((D, H), lambda p, i: (0, 0)),
        pl.BlockSpec((D, H), lambda p, i: (0, 0)),
        pl.BlockSpec((ED, H), lambda p, i: (0, 0)),
        pl.BlockSpec((1, H), lambda p, i: (0, 0)),
        pl.BlockSpec((1, H), lambda p, i: (0, 0)),
        pl.BlockSpec((1, H), lambda p, i: (0, 0)),
    ],
    out_specs=[
        pl.BlockSpec((RB, D), lambda p, i: (jnp.where(p == 1, i, 0), 0)),
        pl.BlockSpec((RB, D), lambda p, i: (jnp.where(p == 1, i, 0), 0)),
        pl.BlockSpec((2, RB, ED), lambda p, i: (0, jnp.where(p == 0, i, 0), 0)),
    ],
    out_shape=(jax.ShapeDtypeStruct((NP, D), jnp.float32),
               jax.ShapeDtypeStruct((NP, D), jnp.float32),
               jax.ShapeDtypeStruct((2, N, ED), jnp.float32)),
    scratch_shapes=[pltpu.VMEM((N, H), jnp.float32),
                    pltpu.VMEM((2, H), jnp.float32)],
)


def _k2_body(h0a_ref, h0b_ref, aggh_ref, aux_ref, w1d_ref, w1s_ref, w1e_ref,
             b1_ref, g_ref, be_ref, wf1_ref, bf1_ref, wf2_ref, bf2_ref,
             out_ref, mp_ref, st_ref, z_ref):
    p = pl.program_id(0)

    @pl.when(p == 0)
    def _():
        i = pl.program_id(1)
        h0 = jnp.concatenate([h0a_ref[...], h0b_ref[...]], axis=1)
        aggh = jnp.concatenate([aggh_ref[0], aggh_ref[1]], axis=1) + h0
        cnt = aux_ref[0][:, 0:1]
        ea = aux_ref[1][...]
        w1e = w1e_ref[...]
        u = jnp.dot(h0, w1d_ref[...], preferred_element_type=jnp.float32) + b1_ref[...]
        pre = (cnt * u
               + jnp.dot(aggh, w1s_ref[...], preferred_element_type=jnp.float32)
               + jnp.dot(ea, w1e, preferred_element_type=jnp.float32)
               + jnp.sum(w1e, axis=0, keepdims=True))
        mp = jnp.maximum(pre, 0.0)
        mp_ref[pl.ds(i * RB, RB)] = mp

        @pl.when(i == 0)
        def _():
            st_ref[...] = jnp.zeros_like(st_ref)
        st_ref[0:1] += jnp.sum(mp, axis=0, keepdims=True)
        st_ref[1:2] += jnp.sum(mp * mp, axis=0, keepdims=True)

    @pl.when(p == 1)
    def _():
        i = pl.program_id(1)
        mp = mp_ref[pl.ds(i * RB, RB)]
        mu = st_ref[0:1] * (1.0 / N)
        var = st_ref[1:2] * (1.0 / N) - mu * mu
        h1 = jnp.maximum((mp - mu) * lax.rsqrt(var + 1e-5) * g_ref[...]
                         + be_ref[...], 0.0)
        h0 = jnp.concatenate([h0a_ref[...], h0b_ref[...]], axis=1)
        col = lax.broadcasted_iota(jnp.int32, (GB, RB), 1)
        row = lax.broadcasted_iota(jnp.int32, (GB, RB), 0)
        seg = jnp.where((col >= row * NPG) & (col < row * NPG + NPG), 1.0, 0.0)
        sel = jnp.where(col == row * NPG, 1.0, 0.0)
        z_ref[i] = jnp.concatenate(
            [jnp.dot(seg, h1, preferred_element_type=jnp.float32),
             jnp.dot(sel, h0, preferred_element_type=jnp.float32),
             jnp.dot(sel, h1, preferred_element_type=jnp.float32)], axis=1)

        @pl.when(i == NG - 1)
        def _():
            zz = jnp.maximum(
                jnp.dot(z_ref[...].reshape(G, 3 * H), wf1_ref[...],
                        preferred_element_type=jnp.float32) + bf1_ref[...], 0.0)
            out_ref[...] = (jnp.dot(zz, wf2_ref[...],
                                    preferred_element_type=jnp.float32)
                            + bf2_ref[...])


_k2 = pl.pallas_call(
    _k2_body,
    grid=(2, NG),
    in_specs=[
        pl.BlockSpec((RB, D), lambda p, i: (i, 0)),
        pl.BlockSpec((RB, D), lambda p, i: (i, 0)),
        pl.BlockSpec((2, RB, D), lambda p, i: (0, jnp.where(p == 0, i, 0), 0)),
        pl.BlockSpec((2, RB, ED), lambda p, i: (0, jnp.where(p == 0, i, 0), 0)),
        pl.BlockSpec((H, H), lambda p, i: (0, 0)),
        pl.BlockSpec((H, H), lambda p, i: (0, 0)),
        pl.BlockSpec((ED, H), lambda p, i: (0, 0)),
        pl.BlockSpec((1, H), lambda p, i: (0, 0)),
        pl.BlockSpec((1, H), lambda p, i: (0, 0)),
        pl.BlockSpec((1, H), lambda p, i: (0, 0)),
        pl.BlockSpec((H + 2 * H, MLP_DIM), lambda p, i: (0, 0)),
        pl.BlockSpec((1, MLP_DIM), lambda p, i: (0, 0)),
        pl.BlockSpec((MLP_DIM, C), lambda p, i: (0, 0)),
        pl.BlockSpec((1, C), lambda p, i: (0, 0)),
    ],
    out_specs=pl.BlockSpec((G, C), lambda p, i: (0, 0)),
    out_shape=jax.ShapeDtypeStruct((G, C), jnp.float32),
    scratch_shapes=[pltpu.VMEM((N, H), jnp.float32),
                    pltpu.VMEM((2, H), jnp.float32),
                    pltpu.VMEM((NG, GB, 3 * H), jnp.float32)],
)


@jax.jit
def kernel(x, edge_index, edge_attr, batch, mask,
           W0, b0, g0, be0, W1, b1, g1, be1, Wf1, bf1, Wf2, bf2):
    ztd = jnp.zeros((RPT, D), jnp.float32)
    zt16 = jnp.zeros((RPT, ED), jnp.float32)
    onesk = jnp.ones((K, ED), jnp.float32)

    aggx, agge, cnt = _make_phase_a()(x, edge_index, edge_attr, ztd, zt16,
                                      onesk)

    h0a, h0b, aux = _k1(x, aggx.reshape(2, NP, D), agge.reshape(2, NP, ED),
                        cnt.reshape(2, NP, ED), W0[0:D], W0[D:2 * D], W0[2 * D:],
                        b0.reshape(1, H), g0.reshape(1, H), be0.reshape(1, H))

    aggh = _make_phase_b()(h0a, h0b, edge_index, ztd)

    return _k2(h0a, h0b, aggh.reshape(2, NP, D), aux,
               W1[0:H], W1[H:2 * H], W1[2 * H:],
               b1.reshape(1, H), g1.reshape(1, H), be1.reshape(1, H),
               Wf1, bf1.reshape(1, MLP_DIM), Wf2, bf2.reshape(1, C))
